# baseline (speedup 1.0000x reference)
import numpy as np
import jax
import jax.numpy as jnp
from jax import lax
from jax.experimental import pallas as pl
from jax.experimental.pallas import tpu as pltpu
from jax.experimental.pallas import tpu_sc as plsc

_B = 8
_C = 192
_N = 224 * 224
_NQ = 16
_L = 16
_G = _C // _L
_TASKS = _B * _G
_NW = 32
_TPW = _TASKS // _NW
_CH = 512
_NCH = _N // _CH
_SHIFTS = (24, 18, 12, 6, 0)
_WIDTHS = (8, 6, 6, 6, 6)
_NR = 2 * _NQ
_HROWS = _NR * 64
_S = 2048
_CH2 = 256
_CAP = _N + _S

_qs = np.arange(1, _NQ + 1, dtype=np.float64) / (_NQ + 1)
_pos = _qs * (_N - 1)
_rlo = np.floor(_pos).astype(np.int64)
_FRAC = (_pos - _rlo).astype(np.float64)
_RANKS = np.stack([_rlo, _rlo + 1], axis=1).reshape(-1)
assert len(set(_RANKS.tolist())) == _NR

_MININT = np.int32(-2147483648)


def _monokey(v):
    i = lax.bitcast_convert_type(v, jnp.int32)
    return i ^ (lax.shift_right_arithmetic(i, 31) | _MININT)


def _keyfloat(u):
    bits = jnp.where(u < 0, u ^ _MININT, ~u)
    return lax.bitcast_convert_type(bits, jnp.float32)


def _zero_i32(ref, nwords):
    z = jnp.zeros((_L,), jnp.int32)

    @plsc.parallel_loop(0, nwords // _L, unroll=8)
    def _(i):
        ref[pl.ds(i * _L, _L)] = z


def _lut_chain(key, lev, lane, luts):
    s = None
    m = None
    for k in range(lev):
        w = _WIDTHS[k]
        d = lax.shift_right_logical(key, _SHIFTS[k]) & ((1 << w) - 1)
        if k == 0:
            v = plsc.load_gather(luts[0], [d * _L + lane])
            m = v > 0
            s = jnp.where(m, v - 1, 0)
        else:
            e = s * 64 + d
            word = plsc.load_gather(
                luts[k], [lax.shift_right_logical(e, 2) * _L + lane])
            bsh = (e & 3) * 8
            v = lax.shift_right_logical(word, bsh) & 255
            m = m & (v > 0)
            s = jnp.where(m, v - 1, 0)
    return s, m


def _body(x_hbm, out_hbm, cbuf, ranks_v, tot_s, hist, lut1w, lut2, lut3,
          lut4, base_a, slot_a, acc_a, totv, buf0, buf1, c2a, c2b, stage,
          ostage, sem0, sem1, sem2):
    luts = (lut1w, lut2, lut3, lut4)
    bufs = (buf0, buf1)
    sems = (sem0, sem1)
    lane = lax.iota(jnp.int32, _L)
    ones16 = jnp.ones((_L,), jnp.int32)
    zeros16 = jnp.zeros((_L,), jnp.int32)

    for j in range(_NR):
        ranks_v[j] = jnp.int32(int(_RANKS[j]))
    wid = lax.axis_index("s") * 2 + lax.axis_index("c")

    def run_task(t, _):
        task = wid * _TPW + t
        b = task // _G
        g = task % _G

        _zero_i32(lut1w, 256 * _L)
        for ref in (lut2, lut3, lut4):
            _zero_i32(ref, 512 * _L)

        def zrow(j, c):
            base_a[j] = zeros16
            slot_a[j] = zeros16
            acc_a[j] = zeros16
            return c

        lax.fori_loop(0, _NR, zrow, 0)
        totv[pl.ds(0, _L)] = zeros16
        for c in range(_L):
            tot_s[c] = jnp.int32(0)

        def flush(w_v, keep_rem):
            if keep_rem:
                fcv = w_v & ~jnp.int32(7)
                rem_v = w_v & jnp.int32(7)
            else:
                fcv = w_v
                rem_v = zeros16
            cps = []
            for c in range(_L):
                off = pl.multiple_of(tot_s[c], 8)
                cp = pltpu.make_async_copy(
                    stage.at[c], cbuf.at[wid, c, pl.ds(off, _S)], sem2)
                cp.start()
                cps.append(cp)
            for cp in cps:
                cp.wait()
            if keep_rem:
                for i in range(8):
                    ii = jnp.int32(i)
                    mv = ii < rem_v
                    src = jnp.minimum(fcv + ii, jnp.int32(_S - 1))
                    vreg = plsc.load_gather(stage, [lane, src], mask=mv)
                    plsc.store_scatter(
                        stage, [lane, jnp.full((_L,), i, jnp.int32)],
                        vreg, mask=mv)
                for c in range(_L):
                    tot_s[c] = tot_s[c] + fcv[c]
            totv[pl.ds(0, _L)] = totv[pl.ds(0, _L)] + fcv
            return rem_v

        for lev in range(5):
            w = _WIDTHS[lev]
            nb = 1 << w
            sh = _SHIFTS[lev]
            nrows = nb if lev == 0 else _HROWS

            _zero_i32(hist, nrows * _L)

            if lev <= 2:
                def dma(ci, hb):
                    return pltpu.make_async_copy(
                        x_hbm.at[b, pl.ds(ci * _CH, _CH), g], bufs[hb],
                        sems[hb])

                def process(hb, w_v):
                    if lev < 2:
                        @plsc.parallel_loop(0, _CH, unroll=8)
                        def _(r):
                            key = _monokey(bufs[hb][r])
                            d = lax.shift_right_logical(key, sh) & (nb - 1)
                            if lev == 0:
                                plsc.addupdate_scatter(
                                    hist, [d * _L + lane], ones16,
                                    mask=lane < _L)
                            else:
                                s, m = _lut_chain(key, lev, lane, luts)
                                idx = (s * 64 + d) * _L + lane
                                plsc.addupdate_scatter(hist, [idx], ones16,
                                                       mask=m)
                        return w_v

                    @plsc.parallel_loop(0, _CH, unroll=8, carry=w_v)
                    def w_out(r, wv):
                        key = _monokey(bufs[hb][r])
                        d = lax.shift_right_logical(key, sh) & (nb - 1)
                        s, m = _lut_chain(key, lev, lane, luts)
                        idx = (s * 64 + d) * _L + lane
                        plsc.addupdate_scatter(hist, [idx], ones16, mask=m)
                        plsc.store_scatter(stage, [lane, wv], key, mask=m)
                        return wv + m.astype(jnp.int32)

                    return w_out

                dma(jnp.int32(0), 0).start()
                dma(jnp.int32(1), 1).start()

                def chunk2(c2, w_v):
                    for hb in (0, 1):
                        ci = 2 * c2 + hb
                        dma(ci, hb).wait()
                        w_v = process(hb, w_v)
                        dma(ci + 2, hb).start()
                    if lev == 2:
                        w_v = lax.cond(
                            jnp.max(w_v) > _S - 2 * _CH,
                            lambda: flush(w_v, True),
                            lambda: w_v)
                    return w_v

                w_v = lax.fori_loop(0, _NCH // 2 - 1, chunk2, zeros16)
                for hb in (0, 1):
                    dma(jnp.int32(_NCH - 2 + hb), hb).wait()
                    w_v = process(hb, w_v)
                if lev == 2:
                    flush(w_v, False)
            else:
                m_v = totv[pl.ds(0, _L)]
                maxm = jnp.max(m_v)
                nch2 = (maxm + _CH2 - 1) // _CH2

                def chunk3(ci, c):
                    cp = pltpu.make_async_copy(
                        cbuf.at[wid, :,
                                pl.ds(pl.multiple_of(ci * _CH2, 8), _CH2)],
                        c2a, sem0)
                    cp.start()
                    cp.wait()
                    base2 = ci * _CH2

                    @plsc.parallel_loop(0, _CH2, unroll=8)
                    def _(r):
                        key = plsc.load_gather(c2a, [lane, zeros16 + r])
                        valid = (base2 + r) < m_v
                        s, m = _lut_chain(key, lev, lane, luts)
                        m = m & valid
                        d = lax.shift_right_logical(key, sh) & (nb - 1)
                        idx = (s * 64 + d) * _L + lane
                        plsc.addupdate_scatter(hist, [idx], ones16, mask=m)

                    return c

                lax.fori_loop(0, nch2, chunk3, 0)

            def cs(i, carry):
                v = hist[pl.ds(i * _L, _L)]
                hist[pl.ds(i * _L, _L)] = carry
                return carry + v

            lax.fori_loop(0, nrows, cs, zeros16)

            steps = []
            st = nb // 2
            while st:
                steps.append(st)
                st //= 2

            def rank_step(j, carry):
                prev_bin, nslots = carry
                r = ranks_v[j]
                lo_entry = zeros16 if lev == 0 else slot_a[j] * nb
                c0 = plsc.load_gather(hist, [lo_entry * _L + lane])
                tgt = (r - base_a[j]) + c0
                p = lo_entry
                for step in steps:
                    cand = p + step
                    cv = plsc.load_gather(hist, [cand * _L + lane])
                    p = jnp.where(cv <= tgt, cand, p)
                cp = plsc.load_gather(hist, [p * _L + lane])
                base_a[j] = base_a[j] + (cp - c0)
                acc_a[j] = lax.shift_left(acc_a[j], w) | (p - lo_entry)
                nslots = nslots + (p != prev_bin).astype(jnp.int32)
                slot_a[j] = nslots - 1
                if lev == 0:
                    plsc.store_scatter(lut1w, [p * _L + lane], nslots)
                elif lev < 4:
                    lutn = luts[lev]
                    wi = lax.shift_right_logical(p, 2) * _L + lane
                    bsh = (p & 3) * 8
                    word = plsc.load_gather(lutn, [wi])
                    word = (word & ~lax.shift_left(jnp.int32(255), bsh)) | \
                        lax.shift_left(nslots, bsh)
                    plsc.store_scatter(lutn, [wi], word)
                return p, nslots

            lax.fori_loop(0, _NR, rank_step,
                          (jnp.full((_L,), -1, jnp.int32), zeros16))

        for i in range(_NQ):
            vlo = _keyfloat(acc_a[2 * i])
            vhi = _keyfloat(acc_a[2 * i + 1])
            f = jnp.float32(_FRAC[i])
            q = vlo * (jnp.float32(1.0) - f) + vhi * f
            plsc.store_scatter(ostage, [lane, jnp.full((_L,), i, jnp.int32)], q)
        pltpu.sync_copy(ostage, out_hbm.at[b, pl.ds(g * _L, _L)])
        return 0

    lax.fori_loop(0, _TPW, run_task, 0)


def kernel(inputs):
    x4 = inputs.reshape(_B, _N, _G, _L)
    mesh = plsc.VectorSubcoreMesh(core_axis_name="c", subcore_axis_name="s")
    fn = pl.kernel(
        _body,
        out_type=jax.ShapeDtypeStruct((_B, _C, _NQ), jnp.float32),
        mesh=mesh,
        compiler_params=pltpu.CompilerParams(
            needs_layout_passes=False, use_tc_tiling_on_sc=False),
        scratch_types=[
            pltpu.HBM((_NW, _L, _CAP), jnp.int32),
            pltpu.SMEM((_NR,), jnp.int32),
            pltpu.SMEM((_L,), jnp.int32),
            pltpu.VMEM((_HROWS * _L,), jnp.int32),
            pltpu.VMEM((256 * _L,), jnp.int32),
            pltpu.VMEM((512 * _L,), jnp.int32),
            pltpu.VMEM((512 * _L,), jnp.int32),
            pltpu.VMEM((512 * _L,), jnp.int32),
            pltpu.VMEM((_NR, _L), jnp.int32),
            pltpu.VMEM((_NR, _L), jnp.int32),
            pltpu.VMEM((_NR, _L), jnp.int32),
            pltpu.VMEM((_L,), jnp.int32),
            pltpu.VMEM((_CH, _L), jnp.float32),
            pltpu.VMEM((_CH, _L), jnp.float32),
            pltpu.VMEM((_L, _CH2), jnp.int32),
            pltpu.VMEM((_L, _CH2), jnp.int32),
            pltpu.VMEM((_L, _S), jnp.int32),
            pltpu.VMEM((_L, _NQ), jnp.float32),
            pltpu.SemaphoreType.DMA,
            pltpu.SemaphoreType.DMA,
            pltpu.SemaphoreType.DMA,
        ],
    )
    return fn(x4)


# speedup vs baseline: 1.0347x; 1.0347x over previous
import numpy as np
import jax
import jax.numpy as jnp
from jax import lax
from jax.experimental import pallas as pl
from jax.experimental.pallas import tpu as pltpu
from jax.experimental.pallas import tpu_sc as plsc

_B = 8
_C = 192
_N = 224 * 224
_NQ = 16
_L = 16
_G = _C // _L
_TASKS = _B * _G
_NW = 32
_TPW = _TASKS // _NW
_CH = 512
_NCH = _N // _CH
_SHIFTS = (24, 18, 12, 6, 0)
_WIDTHS = (8, 6, 6, 6, 6)
_NR = 2 * _NQ
_HROWS = _NR * 64
_S = 2048
_CH2 = 256
_CAP = _N + _S

_qs = np.arange(1, _NQ + 1, dtype=np.float64) / (_NQ + 1)
_pos = _qs * (_N - 1)
_rlo = np.floor(_pos).astype(np.int64)
_FRAC = (_pos - _rlo).astype(np.float64)
_RANKS = np.stack([_rlo, _rlo + 1], axis=1).reshape(-1)
assert len(set(_RANKS.tolist())) == _NR

_MININT = np.int32(-2147483648)


def _monokey(v):
    i = lax.bitcast_convert_type(v, jnp.int32)
    return i ^ (lax.shift_right_arithmetic(i, 31) | _MININT)


def _keyfloat(u):
    bits = jnp.where(u < 0, u ^ _MININT, ~u)
    return lax.bitcast_convert_type(bits, jnp.float32)


def _zero_i32(ref, nwords):
    z = jnp.zeros((_L,), jnp.int32)

    @plsc.parallel_loop(0, nwords // _L, unroll=8)
    def _(i):
        ref[pl.ds(i * _L, _L)] = z


def _lut_chain(key, lev, lane, luts):
    lane_m256 = lane - 256
    d0i = (lax.shift_right_logical(key, 20) & 0xFF0) + lane
    v = plsc.load_gather(luts[1], [d0i])
    m = v > 0
    e = v + (lax.shift_right_logical(key, _SHIFTS[1]) & 63)
    vv = None
    for k in range(1, lev):
        if k > 1:
            e = lax.shift_left(vv, 6) + \
                (lax.shift_right_logical(key, _SHIFTS[k]) & 63)
        wi = (lax.shift_left(lax.shift_right_logical(e, 2), 4)
              + lane_m256) & 0x1FFF
        word = plsc.load_gather(luts[k + 1], [wi])
        bsh = lax.shift_left(e & 3, 3)
        vv = lax.shift_right_logical(word, bsh) & 255
        m = m & (vv > 0)
    return vv, m


def _body(x_hbm, out_hbm, cbuf, ranks_v, tot_s, hist, lut1a, lut1b, lut2,
          lut3, lut4, base_a, slot_a, acc_a, totv, buf0, buf1, c2a, c2b,
          stage, ostage, sem0, sem1, sem2):
    luts = (lut1a, lut1b, lut2, lut3, lut4)
    bufs = (buf0, buf1)
    sems = (sem0, sem1)
    lane = lax.iota(jnp.int32, _L)
    ones16 = jnp.ones((_L,), jnp.int32)
    zeros16 = jnp.zeros((_L,), jnp.int32)

    for j in range(_NR):
        ranks_v[j] = jnp.int32(int(_RANKS[j]))
    wid = lax.axis_index("s") * 2 + lax.axis_index("c")

    def run_task(t, _):
        task = wid * _TPW + t
        b = task // _G
        g = task % _G

        _zero_i32(lut1a, 256 * _L)
        _zero_i32(lut1b, 256 * _L)
        for ref in (lut2, lut3, lut4):
            _zero_i32(ref, 512 * _L)

        def zrow(j, c):
            base_a[j] = zeros16
            slot_a[j] = zeros16
            acc_a[j] = zeros16
            return c

        lax.fori_loop(0, _NR, zrow, 0)
        totv[pl.ds(0, _L)] = zeros16
        for c in range(_L):
            tot_s[c] = jnp.int32(0)

        def flush(w_v, keep_rem):
            if keep_rem:
                fcv = w_v & ~jnp.int32(7)
                rem_v = w_v & jnp.int32(7)
            else:
                fcv = w_v
                rem_v = zeros16
            cps = []
            for c in range(_L):
                off = pl.multiple_of(tot_s[c], 8)
                cp = pltpu.make_async_copy(
                    stage.at[c], cbuf.at[wid, c, pl.ds(off, _S)], sem2)
                cp.start()
                cps.append(cp)
            for cp in cps:
                cp.wait()
            if keep_rem:
                for i in range(8):
                    ii = jnp.int32(i)
                    mv = ii < rem_v
                    src = jnp.minimum(fcv + ii, jnp.int32(_S - 1))
                    vreg = plsc.load_gather(stage, [lane, src], mask=mv)
                    plsc.store_scatter(
                        stage, [lane, jnp.full((_L,), i, jnp.int32)],
                        vreg, mask=mv)
                for c in range(_L):
                    tot_s[c] = tot_s[c] + fcv[c]
            totv[pl.ds(0, _L)] = totv[pl.ds(0, _L)] + fcv
            return rem_v

        for lev in range(5):
            w = _WIDTHS[lev]
            nb = 1 << w
            sh = _SHIFTS[lev]
            nrows = nb if lev == 0 else _HROWS

            _zero_i32(hist, nrows * _L)

            if lev <= 2:
                def dma(ci, hb):
                    return pltpu.make_async_copy(
                        x_hbm.at[b, pl.ds(ci * _CH, _CH), g], bufs[hb],
                        sems[hb])

                lane_m1024 = lane - 1024

                def process(hb, w_v):
                    if lev == 0:
                        @plsc.parallel_loop(0, _CH, unroll=8)
                        def _(r):
                            key = _monokey(bufs[hb][r])
                            idx = (lax.shift_right_logical(key, 20)
                                   & 0xFF0) + lane
                            plsc.addupdate_scatter(hist, [idx], ones16)
                        return w_v

                    if lev == 1:
                        @plsc.parallel_loop(0, _CH, unroll=8)
                        def _(r):
                            key = _monokey(bufs[hb][r])
                            d0i = (lax.shift_right_logical(key, 20)
                                   & 0xFF0) + lane
                            v = plsc.load_gather(lut1a, [d0i])
                            m = v > 0
                            idx = (v + (lax.shift_right_logical(key, 14)
                                        & 0x3F0) + lane_m1024) & 0x7FFF
                            plsc.addupdate_scatter(hist, [idx], ones16,
                                                   mask=m)
                        return w_v

                    @plsc.parallel_loop(0, _CH, unroll=4, carry=w_v)
                    def w_out(r, wv):
                        key = _monokey(bufs[hb][r])
                        vv, m = _lut_chain(key, lev, lane, luts)
                        idx = (lax.shift_left(vv, 10) +
                               (lax.shift_right_logical(key, 8) & 0x3F0) +
                               lane_m1024) & 0x7FFF
                        plsc.addupdate_scatter(hist, [idx], ones16, mask=m)
                        plsc.store_scatter(stage, [lane, wv], key, mask=m)
                        return wv + m.astype(jnp.int32)

                    return w_out

                dma(jnp.int32(0), 0).start()
                dma(jnp.int32(1), 1).start()

                def chunk2(c2, w_v):
                    for hb in (0, 1):
                        ci = 2 * c2 + hb
                        dma(ci, hb).wait()
                        w_v = process(hb, w_v)
                        dma(ci + 2, hb).start()
                    if lev == 2:
                        w_v = lax.cond(
                            jnp.max(w_v) > _S - 2 * _CH,
                            lambda: flush(w_v, True),
                            lambda: w_v)
                    return w_v

                w_v = lax.fori_loop(0, _NCH // 2 - 1, chunk2, zeros16)
                for hb in (0, 1):
                    dma(jnp.int32(_NCH - 2 + hb), hb).wait()
                    w_v = process(hb, w_v)
                if lev == 2:
                    flush(w_v, False)
            else:
                m_v = totv[pl.ds(0, _L)]
                maxm = jnp.max(m_v)
                nch2 = (maxm + _CH2 - 1) // _CH2

                def chunk3(ci, c):
                    cp = pltpu.make_async_copy(
                        cbuf.at[wid, :,
                                pl.ds(pl.multiple_of(ci * _CH2, 8), _CH2)],
                        c2a, sem0)
                    cp.start()
                    cp.wait()
                    base2 = ci * _CH2

                    lane_m1024 = lane - 1024

                    @plsc.parallel_loop(0, _CH2, unroll=8)
                    def _(r):
                        key = plsc.load_gather(c2a, [lane, zeros16 + r])
                        valid = (base2 + r) < m_v
                        vv, m = _lut_chain(key, lev, lane, luts)
                        m = m & valid
                        if sh == 0:
                            d16 = lax.shift_left(key, 4) & 0x3F0
                        else:
                            d16 = lax.shift_right_logical(key, sh - 4) \
                                & 0x3F0
                        idx = (lax.shift_left(vv, 10) + d16
                               + lane_m1024) & 0x7FFF
                        plsc.addupdate_scatter(hist, [idx], ones16, mask=m)

                    return c

                lax.fori_loop(0, nch2, chunk3, 0)

            def cs(i, carry):
                v = hist[pl.ds(i * _L, _L)]
                hist[pl.ds(i * _L, _L)] = carry
                return carry + v

            lax.fori_loop(0, nrows, cs, zeros16)

            steps = []
            st = nb // 2
            while st:
                steps.append(st)
                st //= 2

            def rank_step(j, carry):
                prev_bin, nslots = carry
                r = ranks_v[j]
                lo_entry = zeros16 if lev == 0 else slot_a[j] * nb
                c0 = plsc.load_gather(hist, [lo_entry * _L + lane])
                tgt = (r - base_a[j]) + c0
                p = lo_entry
                for step in steps:
                    cand = p + step
                    cv = plsc.load_gather(hist, [cand * _L + lane])
                    p = jnp.where(cv <= tgt, cand, p)
                cp = plsc.load_gather(hist, [p * _L + lane])
                base_a[j] = base_a[j] + (cp - c0)
                acc_a[j] = lax.shift_left(acc_a[j], w) | (p - lo_entry)
                nslots = nslots + (p != prev_bin).astype(jnp.int32)
                slot_a[j] = nslots - 1
                if lev == 0:
                    wi = p * _L + lane
                    plsc.store_scatter(lut1a, [wi],
                                       lax.shift_left(nslots, 10))
                    plsc.store_scatter(lut1b, [wi],
                                       lax.shift_left(nslots, 6))
                elif lev < 4:
                    lutn = luts[lev + 1]
                    wi = lax.shift_right_logical(p, 2) * _L + lane
                    bsh = (p & 3) * 8
                    word = plsc.load_gather(lutn, [wi])
                    word = (word & ~lax.shift_left(jnp.int32(255), bsh)) | \
                        lax.shift_left(nslots, bsh)
                    plsc.store_scatter(lutn, [wi], word)
                return p, nslots

            lax.fori_loop(0, _NR, rank_step,
                          (jnp.full((_L,), -1, jnp.int32), zeros16))

        for i in range(_NQ):
            vlo = _keyfloat(acc_a[2 * i])
            vhi = _keyfloat(acc_a[2 * i + 1])
            f = jnp.float32(_FRAC[i])
            q = vlo * (jnp.float32(1.0) - f) + vhi * f
            plsc.store_scatter(ostage, [lane, jnp.full((_L,), i, jnp.int32)], q)
        pltpu.sync_copy(ostage, out_hbm.at[b, pl.ds(g * _L, _L)])
        return 0

    lax.fori_loop(0, _TPW, run_task, 0)


def kernel(inputs):
    x4 = inputs.reshape(_B, _N, _G, _L)
    mesh = plsc.VectorSubcoreMesh(core_axis_name="c", subcore_axis_name="s")
    fn = pl.kernel(
        _body,
        out_type=jax.ShapeDtypeStruct((_B, _C, _NQ), jnp.float32),
        mesh=mesh,
        compiler_params=pltpu.CompilerParams(
            needs_layout_passes=False, use_tc_tiling_on_sc=False),
        scratch_types=[
            pltpu.HBM((_NW, _L, _CAP), jnp.int32),
            pltpu.SMEM((_NR,), jnp.int32),
            pltpu.SMEM((_L,), jnp.int32),
            pltpu.VMEM((_HROWS * _L,), jnp.int32),
            pltpu.VMEM((256 * _L,), jnp.int32),
            pltpu.VMEM((256 * _L,), jnp.int32),
            pltpu.VMEM((512 * _L,), jnp.int32),
            pltpu.VMEM((512 * _L,), jnp.int32),
            pltpu.VMEM((512 * _L,), jnp.int32),
            pltpu.VMEM((_NR, _L), jnp.int32),
            pltpu.VMEM((_NR, _L), jnp.int32),
            pltpu.VMEM((_NR, _L), jnp.int32),
            pltpu.VMEM((_L,), jnp.int32),
            pltpu.VMEM((_CH, _L), jnp.float32),
            pltpu.VMEM((_CH, _L), jnp.float32),
            pltpu.VMEM((_L, _CH2), jnp.int32),
            pltpu.VMEM((_L, _CH2), jnp.int32),
            pltpu.VMEM((_L, _S), jnp.int32),
            pltpu.VMEM((_L, _NQ), jnp.float32),
            pltpu.SemaphoreType.DMA,
            pltpu.SemaphoreType.DMA,
            pltpu.SemaphoreType.DMA,
        ],
    )
    return fn(x4)
